# baseline (speedup 1.0000x reference)
import jax
import jax.numpy as jnp
from jax import lax
from jax.experimental import pallas as pl
from jax.experimental.pallas import tpu as pltpu
from jax.experimental.pallas import tpu_sc as plsc

PQ_M = 8
SUB_EMB = 16
VALS_PER_DIM = 256
BATCH = 1024
SEQ_LEN = 200
EMB = PQ_M * SUB_EMB

NC, NS, L = 2, 16, 16
NW = NC * NS
TOTAL = BATCH * SEQ_LEN
PER_W = TOTAL // NW
CHUNK = 128
NCHUNK = PER_W // CHUNK
GROUPS = CHUNK * PQ_M // 128
T_PER_CHUNK = CHUNK // SUB_EMB


def _body(ids_ref, codes_ref, cent_ref, out_ref,
          ids_v, codes_v, flat_v, rows_v,
          sem_ids0, sem_ids1, sem_codes0, sem_codes1,
          sem_rows0, sem_rows1, sem_out0, sem_out1):
    wid = lax.axis_index("s") * NC + lax.axis_index("c")
    sem_ids = (sem_ids0, sem_ids1)
    sem_codes = (sem_codes0, sem_codes1)
    sem_rows = (sem_rows0, sem_rows1)
    sem_out = (sem_out0, sem_out1)

    iota = lax.iota(jnp.int32, L)
    row_pat = iota // PQ_M
    col_pat = lax.rem(iota, PQ_M)
    off_pat = col_pat * VALS_PER_DIM

    def t_base(k):
        return wid * (PER_W // SUB_EMB) + k * T_PER_CHUNK

    def fire_ids(k, p):
        pltpu.async_copy(ids_ref.at[wid * NCHUNK + k], ids_v.at[p],
                         sem_ids[p])

    def wait_ids(p):
        pltpu.make_async_copy(ids_ref.at[0], ids_v.at[p], sem_ids[p]).wait()

    def fire_codes(p):
        pltpu.async_copy(codes_ref.at[ids_v.at[p]], codes_v.at[p],
                         sem_codes[p])

    def wait_codes(p):
        pltpu.make_async_copy(codes_ref.at[pl.ds(0, CHUNK)], codes_v.at[p],
                              sem_codes[p]).wait()

    def flat_compute(p):
        def idx_body(t, c):
            rows16 = row_pat + 2 * t
            codes16 = plsc.load_gather(codes_v.at[p], [rows16, col_pat])
            g = t // 8
            o = (t - g * 8) * L
            flat_v.at[p].at[g][pl.ds(o, L)] = (codes16 & 0) + off_pat
            return c

        lax.fori_loop(0, CHUNK * PQ_M // L, idx_body, 0, unroll=8)

    def fire_gathers(p):
        for g in range(GROUPS):
            pltpu.async_copy(cent_ref.at[flat_v.at[p].at[g]],
                             rows_v.at[p].at[g], sem_rows[p])

    def drain_gathers(p):
        pltpu.make_async_copy(out_ref.at[pl.ds(0, T_PER_CHUNK)],
                              rows_v.at[p], sem_rows[p]).wait()

    def fire_out(k, p):
        pltpu.async_copy(rows_v.at[p],
                         out_ref.at[pl.ds(t_base(k), T_PER_CHUNK)],
                         sem_out[p])

    def drain_out(p):
        pltpu.make_async_copy(rows_v.at[p],
                              out_ref.at[pl.ds(0, T_PER_CHUNK)],
                              sem_out[p]).wait()

    fire_ids(0, 0)
    fire_ids(1, 1)
    wait_ids(0)
    fire_codes(0)

    def pair_body(kk, carry):
        for p in (0, 1):
            k = 2 * kk + p

            wait_codes(p)

            @pl.when(k < NCHUNK - 2)
            def _():
                fire_ids(k + 2, p)

            flat_compute(p)

            @pl.when(k >= 2)
            def _():
                drain_out(p)

            @pl.when(k >= 1)
            def _():
                drain_gathers(1 - p)
                fire_out(k - 1, 1 - p)

            fire_gathers(p)

            @pl.when(k < NCHUNK - 1)
            def _():
                wait_ids(1 - p)
                fire_codes(1 - p)
        return carry

    lax.fori_loop(0, NCHUNK // 2, pair_body, 0)
    drain_gathers(1)
    fire_out(NCHUNK - 1, 1)
    drain_out(0)
    drain_out(1)


@jax.jit
def _sc_call(ids2d, item_codes, cent2d):
    mesh = plsc.VectorSubcoreMesh(core_axis_name="c", subcore_axis_name="s")
    f = pl.kernel(
        _body,
        out_type=jax.ShapeDtypeStruct((TOTAL // SUB_EMB, 128, SUB_EMB),
                                      jnp.float32),
        mesh=mesh,
        scratch_types=[
            pltpu.VMEM((2, CHUNK), jnp.int32),
            pltpu.VMEM((2, CHUNK, PQ_M), jnp.int32),
            pltpu.VMEM((2, GROUPS, 128), jnp.int32),
            pltpu.VMEM((2, GROUPS, 128, SUB_EMB), jnp.float32),
            pltpu.SemaphoreType.DMA,
            pltpu.SemaphoreType.DMA,
            pltpu.SemaphoreType.DMA,
            pltpu.SemaphoreType.DMA,
            pltpu.SemaphoreType.DMA,
            pltpu.SemaphoreType.DMA,
            pltpu.SemaphoreType.DMA,
            pltpu.SemaphoreType.DMA,
        ],
        compiler_params=pltpu.CompilerParams(use_tc_tiling_on_sc=False,
                                             needs_layout_passes=False),
    )
    return f(ids2d, item_codes, cent2d)


def kernel(input_ids, item_codes, centroids):
    ids2d = input_ids.reshape(TOTAL // 128, 128)
    cent2d = centroids.reshape(PQ_M * VALS_PER_DIM, SUB_EMB)
    out3d = _sc_call(ids2d, item_codes, cent2d)
    return out3d.reshape(BATCH, SEQ_LEN, EMB)


# speedup vs baseline: 1.9014x; 1.9014x over previous
import jax
import jax.numpy as jnp
from jax import lax
from jax.experimental import pallas as pl
from jax.experimental.pallas import tpu as pltpu
from jax.experimental.pallas import tpu_sc as plsc

PQ_M = 8
SUB_EMB = 16
VALS_PER_DIM = 256
BATCH = 1024
SEQ_LEN = 200
EMB = PQ_M * SUB_EMB

NC, NS, L = 2, 16, 16
NW = NC * NS
TOTAL = BATCH * SEQ_LEN
PER_W = TOTAL // NW
CHUNK = 128
NCHUNK = PER_W // CHUNK
CENT_WORDS = PQ_M * VALS_PER_DIM * SUB_EMB
CHUNK_WORDS = CHUNK * EMB


def _body(ids_ref, codes_ref, cent_ref, out_ref,
          ids_v, codes_v, rows_v, cent_v,
          sem_cent, sem_ids0, sem_ids1, sem_codes0, sem_codes1,
          sem_out0, sem_out1):
    wid = lax.axis_index("s") * NC + lax.axis_index("c")
    sem_ids = (sem_ids0, sem_ids1)
    sem_codes = (sem_codes0, sem_codes1)
    sem_out = (sem_out0, sem_out1)

    iota = lax.iota(jnp.int32, L)
    row_pat = iota // PQ_M
    col_pat = lax.rem(iota, PQ_M)
    moff_pat = col_pat * (VALS_PER_DIM * SUB_EMB)
    iota16 = iota * SUB_EMB

    def fire_ids(k, p):
        pltpu.async_copy(ids_ref.at[wid * NCHUNK + k], ids_v.at[p],
                         sem_ids[p])

    def wait_ids(p):
        pltpu.make_async_copy(ids_ref.at[0], ids_v.at[p], sem_ids[p]).wait()

    def fire_codes(p):
        pltpu.async_copy(codes_ref.at[ids_v.at[p]], codes_v.at[p],
                         sem_codes[p])

    def wait_codes(p):
        pltpu.make_async_copy(codes_ref.at[pl.ds(0, CHUNK)], codes_v.at[p],
                              sem_codes[p]).wait()

    def compute_chunk(p):
        def t_body(t, c):
            rows16 = row_pat + 2 * t
            codes16 = plsc.load_gather(codes_v.at[p], [rows16, col_pat])
            flat16 = codes16 * SUB_EMB + moff_pat
            sbase = iota16 + 256 * t
            for j in range(SUB_EMB):
                vals = plsc.load_gather(cent_v, [flat16 + j])
                plsc.store_scatter(rows_v.at[p], [sbase + j], vals)
            return c

        lax.fori_loop(0, CHUNK * PQ_M // L, t_body, 0, unroll=4)

    def fire_out(k, p):
        pltpu.async_copy(rows_v.at[p], out_ref.at[wid * NCHUNK + k],
                         sem_out[p])

    def drain_out(p):
        pltpu.make_async_copy(rows_v.at[p], out_ref.at[0], sem_out[p]).wait()

    pltpu.async_copy(cent_ref, cent_v, sem_cent)
    fire_ids(0, 0)
    fire_ids(1, 1)
    wait_ids(0)
    fire_codes(0)
    pltpu.make_async_copy(cent_ref, cent_v, sem_cent).wait()

    def pair_body(kk, carry):
        for p in (0, 1):
            k = 2 * kk + p

            wait_codes(p)

            @pl.when(k < NCHUNK - 2)
            def _():
                fire_ids(k + 2, p)

            @pl.when(k >= 2)
            def _():
                drain_out(p)

            compute_chunk(p)
            fire_out(k, p)

            @pl.when(k < NCHUNK - 1)
            def _():
                wait_ids(1 - p)
                fire_codes(1 - p)
        return carry

    lax.fori_loop(0, NCHUNK // 2, pair_body, 0)
    drain_out(0)
    drain_out(1)


@jax.jit
def _sc_call(ids2d, item_codes, cent1d):
    mesh = plsc.VectorSubcoreMesh(core_axis_name="c", subcore_axis_name="s")
    f = pl.kernel(
        _body,
        out_type=jax.ShapeDtypeStruct((TOTAL * EMB // CHUNK_WORDS,
                                       CHUNK_WORDS), jnp.float32),
        mesh=mesh,
        scratch_types=[
            pltpu.VMEM((2, CHUNK), jnp.int32),
            pltpu.VMEM((2, CHUNK, PQ_M), jnp.int32),
            pltpu.VMEM((2, CHUNK_WORDS), jnp.float32),
            pltpu.VMEM((CENT_WORDS,), jnp.float32),
            pltpu.SemaphoreType.DMA,
            pltpu.SemaphoreType.DMA,
            pltpu.SemaphoreType.DMA,
            pltpu.SemaphoreType.DMA,
            pltpu.SemaphoreType.DMA,
            pltpu.SemaphoreType.DMA,
            pltpu.SemaphoreType.DMA,
        ],
        compiler_params=pltpu.CompilerParams(use_tc_tiling_on_sc=False,
                                             needs_layout_passes=False),
    )
    return f(ids2d, item_codes, cent1d)


def kernel(input_ids, item_codes, centroids):
    ids2d = input_ids.reshape(TOTAL // CHUNK, CHUNK)
    cent1d = centroids.reshape(CENT_WORDS)
    out2d = _sc_call(ids2d, item_codes, cent1d)
    return out2d.reshape(BATCH, SEQ_LEN, EMB)


# speedup vs baseline: 7.1842x; 3.7784x over previous
import jax
import jax.numpy as jnp
from jax import lax
from jax.experimental import pallas as pl
from jax.experimental.pallas import tpu as pltpu
from jax.experimental.pallas import tpu_sc as plsc

PQ_M = 8
SUB_EMB = 16
VALS_PER_DIM = 256
BATCH = 1024
SEQ_LEN = 200
EMB = PQ_M * SUB_EMB

NC, NS, L = 2, 16, 16
NW = NC * NS
TOTAL = BATCH * SEQ_LEN
PER_W = TOTAL // NW
CHUNK = 128
NCHUNK = PER_W // CHUNK
GROUPS = CHUNK * PQ_M // 128
T_PER_CHUNK = CHUNK // SUB_EMB


def _body(ids_ref, codes_ref, cent_ref, out_ref,
          ids_v, codes_v, flat_v, rows_v, cent_sh,
          sem_cent, sem_ids0, sem_ids1, sem_codes0, sem_codes1,
          sem_rows0, sem_rows1, sem_out0, sem_out1):
    wid = lax.axis_index("s") * NC + lax.axis_index("c")
    sem_ids = (sem_ids0, sem_ids1)
    sem_codes = (sem_codes0, sem_codes1)
    sem_rows = (sem_rows0, sem_rows1)
    sem_out = (sem_out0, sem_out1)

    iota = lax.iota(jnp.int32, L)
    row_pat = iota // PQ_M
    col_pat = lax.rem(iota, PQ_M)
    off_pat = col_pat * VALS_PER_DIM

    def t_base(k):
        return wid * (PER_W // SUB_EMB) + k * T_PER_CHUNK

    def fire_ids(k, p):
        pltpu.async_copy(ids_ref.at[wid * NCHUNK + k], ids_v.at[p],
                         sem_ids[p])

    def wait_ids(p):
        pltpu.make_async_copy(ids_ref.at[0], ids_v.at[p], sem_ids[p]).wait()

    def fire_codes(p):
        pltpu.async_copy(codes_ref.at[ids_v.at[p]], codes_v.at[p],
                         sem_codes[p])

    def wait_codes(p):
        pltpu.make_async_copy(codes_ref.at[pl.ds(0, CHUNK)], codes_v.at[p],
                              sem_codes[p]).wait()

    def flat_compute(p):
        def idx_body(t, c):
            rows16 = row_pat + 2 * t
            codes16 = plsc.load_gather(codes_v.at[p], [rows16, col_pat])
            g = t // 8
            o = (t - g * 8) * L
            flat_v.at[p].at[g][pl.ds(o, L)] = codes16 + off_pat
            return c

        lax.fori_loop(0, CHUNK * PQ_M // L, idx_body, 0, unroll=8)

    def fire_gathers(p):
        for g in range(GROUPS):
            pltpu.async_copy(cent_sh.at[flat_v.at[p].at[g]],
                             rows_v.at[p].at[g], sem_rows[p])

    def drain_gathers(p):
        pltpu.make_async_copy(out_ref.at[pl.ds(0, T_PER_CHUNK)],
                              rows_v.at[p], sem_rows[p]).wait()

    def fire_out(k, p):
        pltpu.async_copy(rows_v.at[p],
                         out_ref.at[pl.ds(t_base(k), T_PER_CHUNK)],
                         sem_out[p])

    def drain_out(p):
        pltpu.make_async_copy(rows_v.at[p],
                              out_ref.at[pl.ds(0, T_PER_CHUNK)],
                              sem_out[p]).wait()

    @pl.when(lax.axis_index("s") == 0)
    def _():
        pltpu.async_copy(cent_ref, cent_sh, sem_cent).wait()
    plsc.subcore_barrier()
    fire_ids(0, 0)
    fire_ids(1, 1)
    wait_ids(0)
    fire_codes(0)

    def pair_body(kk, carry):
        for p in (0, 1):
            k = 2 * kk + p

            wait_codes(p)

            @pl.when(k < NCHUNK - 2)
            def _():
                fire_ids(k + 2, p)

            flat_compute(p)

            @pl.when(k >= 2)
            def _():
                drain_out(p)

            @pl.when(k >= 1)
            def _():
                drain_gathers(1 - p)
                fire_out(k - 1, 1 - p)

            fire_gathers(p)

            @pl.when(k < NCHUNK - 1)
            def _():
                wait_ids(1 - p)
                fire_codes(1 - p)
        return carry

    lax.fori_loop(0, NCHUNK // 2, pair_body, 0)
    drain_gathers(1)
    fire_out(NCHUNK - 1, 1)
    drain_out(0)
    drain_out(1)


@jax.jit
def _sc_call(ids2d, item_codes, cent2d):
    mesh = plsc.VectorSubcoreMesh(core_axis_name="c", subcore_axis_name="s")
    f = pl.kernel(
        _body,
        out_type=jax.ShapeDtypeStruct((TOTAL // SUB_EMB, 128, SUB_EMB),
                                      jnp.float32),
        mesh=mesh,
        scratch_types=[
            pltpu.VMEM((2, CHUNK), jnp.int32),
            pltpu.VMEM((2, CHUNK, PQ_M), jnp.int32),
            pltpu.VMEM((2, GROUPS, 128), jnp.int32),
            pltpu.VMEM((2, GROUPS, 128, SUB_EMB), jnp.float32),
            pltpu.VMEM_SHARED((PQ_M * VALS_PER_DIM, SUB_EMB), jnp.float32),
            pltpu.SemaphoreType.DMA,
            pltpu.SemaphoreType.DMA,
            pltpu.SemaphoreType.DMA,
            pltpu.SemaphoreType.DMA,
            pltpu.SemaphoreType.DMA,
            pltpu.SemaphoreType.DMA,
            pltpu.SemaphoreType.DMA,
            pltpu.SemaphoreType.DMA,
            pltpu.SemaphoreType.DMA,
        ],
        compiler_params=pltpu.CompilerParams(use_tc_tiling_on_sc=False,
                                             needs_layout_passes=False),
    )
    return f(ids2d, item_codes, cent2d)


def kernel(input_ids, item_codes, centroids):
    ids2d = input_ids.reshape(TOTAL // 128, 128)
    cent2d = centroids.reshape(PQ_M * VALS_PER_DIM, SUB_EMB)
    out3d = _sc_call(ids2d, item_codes, cent2d)
    return out3d.reshape(BATCH, SEQ_LEN, EMB)


# speedup vs baseline: 7.1899x; 1.0008x over previous
import jax
import jax.numpy as jnp
from jax import lax
from jax.experimental import pallas as pl
from jax.experimental.pallas import tpu as pltpu
from jax.experimental.pallas import tpu_sc as plsc

PQ_M = 8
SUB_EMB = 16
VALS_PER_DIM = 256
BATCH = 1024
SEQ_LEN = 200
EMB = PQ_M * SUB_EMB

NC, NS, L = 2, 16, 16
NW = NC * NS
TOTAL = BATCH * SEQ_LEN
PER_W = TOTAL // NW
CHUNK = 128
NCHUNK = PER_W // CHUNK
GROUPS = CHUNK * PQ_M // 128
T_PER_CHUNK = CHUNK // SUB_EMB


def _body(ids_ref, codes_ref, cent_ref, out_ref,
          ids_v, codes_v, flat_v, rows_v, cent_sh,
          sem_cent, sem_ids0, sem_ids1, sem_codes0, sem_codes1,
          sem_rows0, sem_rows1, sem_out0, sem_out1):
    wid = lax.axis_index("s") * NC + lax.axis_index("c")
    sem_ids = (sem_ids0, sem_ids1)
    sem_codes = (sem_codes0, sem_codes1)
    sem_rows = (sem_rows0, sem_rows1)
    sem_out = (sem_out0, sem_out1)

    iota = lax.iota(jnp.int32, L)
    row_pat = iota // PQ_M
    col_pat = lax.rem(iota, PQ_M)
    off_pat = col_pat * VALS_PER_DIM

    def t_base(k):
        return wid * (PER_W // SUB_EMB) + k * T_PER_CHUNK

    def fire_ids(k, p):
        pltpu.async_copy(ids_ref.at[pl.ds((wid * NCHUNK + k) * CHUNK, CHUNK)],
                         ids_v.at[p], sem_ids[p])

    def wait_ids(p):
        pltpu.make_async_copy(ids_ref.at[pl.ds(0, CHUNK)], ids_v.at[p],
                              sem_ids[p]).wait()

    def fire_codes(p):
        pltpu.async_copy(codes_ref.at[ids_v.at[p]], codes_v.at[p],
                         sem_codes[p])

    def wait_codes(p):
        pltpu.make_async_copy(codes_ref.at[pl.ds(0, CHUNK)], codes_v.at[p],
                              sem_codes[p]).wait()

    def flat_compute(p):
        def idx_body(t, c):
            rows16 = row_pat + 2 * t
            codes16 = plsc.load_gather(codes_v.at[p], [rows16, col_pat])
            g = t // 8
            o = (t - g * 8) * L
            flat_v.at[p].at[g][pl.ds(o, L)] = codes16 + off_pat
            return c

        lax.fori_loop(0, CHUNK * PQ_M // L, idx_body, 0, unroll=8)

    def fire_gathers(p):
        for g in range(GROUPS):
            pltpu.async_copy(cent_sh.at[flat_v.at[p].at[g]],
                             rows_v.at[p].at[g], sem_rows[p])

    def drain_gathers(p):
        pltpu.make_async_copy(out_ref.at[pl.ds(0, T_PER_CHUNK)],
                              rows_v.at[p], sem_rows[p]).wait()

    def fire_out(k, p):
        pltpu.async_copy(rows_v.at[p],
                         out_ref.at[pl.ds(t_base(k), T_PER_CHUNK)],
                         sem_out[p])

    def drain_out(p):
        pltpu.make_async_copy(rows_v.at[p],
                              out_ref.at[pl.ds(0, T_PER_CHUNK)],
                              sem_out[p]).wait()

    @pl.when(lax.axis_index("s") == 0)
    def _():
        pltpu.async_copy(cent_ref, cent_sh, sem_cent).wait()
    plsc.subcore_barrier()
    fire_ids(0, 0)
    fire_ids(1, 1)
    wait_ids(0)
    fire_codes(0)

    def pair_body(kk, carry):
        for p in (0, 1):
            k = 2 * kk + p

            wait_codes(p)

            @pl.when(k < NCHUNK - 2)
            def _():
                fire_ids(k + 2, p)

            flat_compute(p)

            @pl.when(k >= 2)
            def _():
                drain_out(p)

            @pl.when(k >= 1)
            def _():
                drain_gathers(1 - p)
                fire_out(k - 1, 1 - p)

            fire_gathers(p)

            @pl.when(k < NCHUNK - 1)
            def _():
                wait_ids(1 - p)
                fire_codes(1 - p)
        return carry

    lax.fori_loop(0, NCHUNK // 2, pair_body, 0)
    drain_gathers(1)
    fire_out(NCHUNK - 1, 1)
    drain_out(0)
    drain_out(1)


@jax.jit
def _sc_call(ids2d, item_codes, cent2d):
    mesh = plsc.VectorSubcoreMesh(core_axis_name="c", subcore_axis_name="s")
    f = pl.kernel(
        _body,
        out_type=jax.ShapeDtypeStruct((TOTAL // SUB_EMB, 128, SUB_EMB),
                                      jnp.float32),
        mesh=mesh,
        scratch_types=[
            pltpu.VMEM((2, CHUNK), jnp.int32),
            pltpu.VMEM((2, CHUNK, PQ_M), jnp.int32),
            pltpu.VMEM((2, GROUPS, 128), jnp.int32),
            pltpu.VMEM((2, GROUPS, 128, SUB_EMB), jnp.float32),
            pltpu.VMEM_SHARED((PQ_M * VALS_PER_DIM, SUB_EMB), jnp.float32),
            pltpu.SemaphoreType.DMA,
            pltpu.SemaphoreType.DMA,
            pltpu.SemaphoreType.DMA,
            pltpu.SemaphoreType.DMA,
            pltpu.SemaphoreType.DMA,
            pltpu.SemaphoreType.DMA,
            pltpu.SemaphoreType.DMA,
            pltpu.SemaphoreType.DMA,
            pltpu.SemaphoreType.DMA,
        ],
        compiler_params=pltpu.CompilerParams(use_tc_tiling_on_sc=False,
                                             needs_layout_passes=False),
    )
    return f(ids2d, item_codes, cent2d)


def kernel(input_ids, item_codes, centroids):
    ids1d = input_ids.reshape(TOTAL)
    cent2d = centroids.reshape(PQ_M * VALS_PER_DIM, SUB_EMB)
    out3d = _sc_call(ids1d, item_codes, cent2d)
    return out3d.reshape(BATCH, SEQ_LEN, EMB)


# speedup vs baseline: 8.6977x; 1.2097x over previous
import jax
import jax.numpy as jnp
from jax import lax
from jax.experimental import pallas as pl
from jax.experimental.pallas import tpu as pltpu
from jax.experimental.pallas import tpu_sc as plsc

PQ_M = 8
SUB_EMB = 16
VALS_PER_DIM = 256
BATCH = 1024
SEQ_LEN = 200
EMB = PQ_M * SUB_EMB

NC, NS, L = 2, 16, 16
NW = NC * NS
TOTAL = BATCH * SEQ_LEN
PER_W = TOTAL // NW
CHUNK = 128
NCHUNK = PER_W // CHUNK
GROUPS = CHUNK * PQ_M // 128
T_PER_CHUNK = CHUNK // SUB_EMB


def _body(ids_ref, codes0_ref, codes1_ref, cent_ref, out_ref,
          ids_v, codes0_v, codes1_v, flat_v, rows_v, cent_sh,
          sem_cent, sem_ids0, sem_ids1, sem_codes0, sem_codes1,
          sem_rows0, sem_rows1, sem_out0, sem_out1):
    wid = lax.axis_index("s") * NC + lax.axis_index("c")
    sem_ids = (sem_ids0, sem_ids1)
    sem_codes = (sem_codes0, sem_codes1)
    sem_rows = (sem_rows0, sem_rows1)
    sem_out = (sem_out0, sem_out1)

    iota = lax.iota(jnp.int32, L)
    row_pat = iota // PQ_M
    col_pat = lax.rem(iota, PQ_M)
    off_pat = col_pat * VALS_PER_DIM
    lo_mask = col_pat < 4
    shift_pat = lax.rem(iota, 4) * 8

    def t_base(k):
        return wid * (PER_W // SUB_EMB) + k * T_PER_CHUNK

    def fire_ids(k, p):
        pltpu.async_copy(ids_ref.at[pl.ds((wid * NCHUNK + k) * CHUNK, CHUNK)],
                         ids_v.at[p], sem_ids[p])

    def wait_ids(p):
        pltpu.make_async_copy(ids_ref.at[pl.ds(0, CHUNK)], ids_v.at[p],
                              sem_ids[p]).wait()

    def fire_codes(p):
        pltpu.async_copy(codes0_ref.at[ids_v.at[p]], codes0_v.at[p],
                         sem_codes[p])
        pltpu.async_copy(codes1_ref.at[ids_v.at[p]], codes1_v.at[p],
                         sem_codes[p])

    def wait_codes(p):
        pltpu.make_async_copy(codes0_ref.at[pl.ds(0, CHUNK)], codes0_v.at[p],
                              sem_codes[p]).wait()
        pltpu.make_async_copy(codes1_ref.at[pl.ds(0, CHUNK)], codes1_v.at[p],
                              sem_codes[p]).wait()

    def flat_compute(p):
        def idx_body(t, c):
            rows16 = row_pat + 2 * t
            w0 = plsc.load_gather(codes0_v.at[p], [rows16])
            w1 = plsc.load_gather(codes1_v.at[p], [rows16])
            words16 = lax.select(lo_mask, w0, w1)
            codes16 = lax.shift_right_logical(words16, shift_pat) & 255
            g = t // 8
            o = (t - g * 8) * L
            flat_v.at[p].at[g][pl.ds(o, L)] = codes16 + off_pat
            return c

        lax.fori_loop(0, CHUNK * PQ_M // L, idx_body, 0, unroll=8)

    def fire_gathers(p):
        for g in range(GROUPS):
            pltpu.async_copy(cent_sh.at[flat_v.at[p].at[g]],
                             rows_v.at[p].at[g], sem_rows[p])

    def drain_gathers(p):
        pltpu.make_async_copy(out_ref.at[pl.ds(0, T_PER_CHUNK)],
                              rows_v.at[p], sem_rows[p]).wait()

    def fire_out(k, p):
        pltpu.async_copy(rows_v.at[p],
                         out_ref.at[pl.ds(t_base(k), T_PER_CHUNK)],
                         sem_out[p])

    def drain_out(p):
        pltpu.make_async_copy(rows_v.at[p],
                              out_ref.at[pl.ds(0, T_PER_CHUNK)],
                              sem_out[p]).wait()

    @pl.when(lax.axis_index("s") == 0)
    def _():
        pltpu.async_copy(cent_ref, cent_sh, sem_cent).wait()
    plsc.subcore_barrier()
    fire_ids(0, 0)
    fire_ids(1, 1)
    wait_ids(0)
    fire_codes(0)

    def pair_body(kk, carry):
        for p in (0, 1):
            k = 2 * kk + p

            wait_codes(p)

            @pl.when(k < NCHUNK - 2)
            def _():
                fire_ids(k + 2, p)

            flat_compute(p)

            @pl.when(k >= 2)
            def _():
                drain_out(p)

            @pl.when(k >= 1)
            def _():
                drain_gathers(1 - p)
                fire_out(k - 1, 1 - p)

            fire_gathers(p)

            @pl.when(k < NCHUNK - 1)
            def _():
                wait_ids(1 - p)
                fire_codes(1 - p)
        return carry

    lax.fori_loop(0, NCHUNK // 2, pair_body, 0)
    drain_gathers(1)
    fire_out(NCHUNK - 1, 1)
    drain_out(0)
    drain_out(1)


@jax.jit
def _sc_call(ids1d, codes0, codes1, cent2d):
    mesh = plsc.VectorSubcoreMesh(core_axis_name="c", subcore_axis_name="s")
    f = pl.kernel(
        _body,
        out_type=jax.ShapeDtypeStruct((TOTAL // SUB_EMB, 128, SUB_EMB),
                                      jnp.float32),
        mesh=mesh,
        scratch_types=[
            pltpu.VMEM((2, CHUNK), jnp.int32),
            pltpu.VMEM((2, CHUNK), jnp.int32),
            pltpu.VMEM((2, CHUNK), jnp.int32),
            pltpu.VMEM((2, GROUPS, 128), jnp.int32),
            pltpu.VMEM((2, GROUPS, 128, SUB_EMB), jnp.float32),
            pltpu.VMEM_SHARED((PQ_M * VALS_PER_DIM, SUB_EMB), jnp.float32),
            pltpu.SemaphoreType.DMA,
            pltpu.SemaphoreType.DMA,
            pltpu.SemaphoreType.DMA,
            pltpu.SemaphoreType.DMA,
            pltpu.SemaphoreType.DMA,
            pltpu.SemaphoreType.DMA,
            pltpu.SemaphoreType.DMA,
            pltpu.SemaphoreType.DMA,
            pltpu.SemaphoreType.DMA,
        ],
        compiler_params=pltpu.CompilerParams(use_tc_tiling_on_sc=False,
                                             needs_layout_passes=False),
    )
    return f(ids1d, codes0, codes1, cent2d)


def kernel(input_ids, item_codes, centroids):
    ids1d = input_ids.reshape(TOTAL)
    cent2d = centroids.reshape(PQ_M * VALS_PER_DIM, SUB_EMB)
    c = item_codes
    w0 = c[:, 0] | (c[:, 1] << 8) | (c[:, 2] << 16) | (c[:, 3] << 24)
    w1 = c[:, 4] | (c[:, 5] << 8) | (c[:, 6] << 16) | (c[:, 7] << 24)
    out3d = _sc_call(ids1d, w0, w1, cent2d)
    return out3d.reshape(BATCH, SEQ_LEN, EMB)


# speedup vs baseline: 13.2076x; 1.5185x over previous
import jax
import jax.numpy as jnp
from jax import lax
from jax.experimental import pallas as pl
from jax.experimental.pallas import tpu as pltpu
from jax.experimental.pallas import tpu_sc as plsc

PQ_M = 8
SUB_EMB = 16
VALS_PER_DIM = 256
BATCH = 1024
SEQ_LEN = 200
EMB = PQ_M * SUB_EMB

NC, NS, L = 2, 16, 16
NW = NC * NS
TOTAL = BATCH * SEQ_LEN
PER_W = TOTAL // NW
CHUNK = 128
NCHUNK = PER_W // CHUNK
GROUPS = CHUNK * PQ_M // 128
T_PER_CHUNK = CHUNK // SUB_EMB


def _body(ids_ref, codes0_ref, codes1_ref, cent_ref, out_ref,
          ids_v, codes0_v, codes1_v, flat_v, rows_v, cent_sh, codes_sh,
          sem_cent, sem_ids0, sem_ids1, sem_codes0, sem_codes1,
          sem_rows0, sem_rows1, sem_out0, sem_out1):
    wid = lax.axis_index("s") * NC + lax.axis_index("c")
    sem_ids = (sem_ids0, sem_ids1)
    sem_codes = (sem_codes0, sem_codes1)
    sem_rows = (sem_rows0, sem_rows1)
    sem_out = (sem_out0, sem_out1)

    iota = lax.iota(jnp.int32, L)
    row_pat = iota // PQ_M
    col_pat = lax.rem(iota, PQ_M)
    off_pat = col_pat * VALS_PER_DIM
    lo_mask = col_pat < 4
    shift_pat = lax.rem(iota, 4) * 8

    def t_base(k):
        return wid * (PER_W // SUB_EMB) + k * T_PER_CHUNK

    def fire_ids(k, p):
        pltpu.async_copy(ids_ref.at[pl.ds((wid * NCHUNK + k) * CHUNK, CHUNK)],
                         ids_v.at[p], sem_ids[p])

    def wait_ids(p):
        pltpu.make_async_copy(ids_ref.at[pl.ds(0, CHUNK)], ids_v.at[p],
                              sem_ids[p]).wait()

    def fire_codes(p):
        pltpu.async_copy(codes_sh.at[0].at[ids_v.at[p]], codes0_v.at[p],
                         sem_codes[p])
        pltpu.async_copy(codes_sh.at[1].at[ids_v.at[p]], codes1_v.at[p],
                         sem_codes[p])

    def wait_codes(p):
        pltpu.make_async_copy(codes0_ref.at[pl.ds(0, CHUNK)], codes0_v.at[p],
                              sem_codes[p]).wait()
        pltpu.make_async_copy(codes1_ref.at[pl.ds(0, CHUNK)], codes1_v.at[p],
                              sem_codes[p]).wait()

    def flat_compute(p):
        def idx_body(t, c):
            rows16 = row_pat + 2 * t
            w0 = plsc.load_gather(codes0_v.at[p], [rows16])
            w1 = plsc.load_gather(codes1_v.at[p], [rows16])
            words16 = lax.select(lo_mask, w0, w1)
            codes16 = lax.shift_right_logical(words16, shift_pat) & 255
            g = t // 8
            o = (t - g * 8) * L
            flat_v.at[p].at[g][pl.ds(o, L)] = codes16 + off_pat
            return c

        lax.fori_loop(0, CHUNK * PQ_M // L, idx_body, 0, unroll=8)

    def fire_gathers(p):
        for g in range(GROUPS):
            pltpu.async_copy(cent_sh.at[flat_v.at[p].at[g]],
                             rows_v.at[p].at[g], sem_rows[p])

    def drain_gathers(p):
        pltpu.make_async_copy(out_ref.at[pl.ds(0, T_PER_CHUNK)],
                              rows_v.at[p], sem_rows[p]).wait()

    def fire_out(k, p):
        pltpu.async_copy(rows_v.at[p],
                         out_ref.at[pl.ds(t_base(k), T_PER_CHUNK)],
                         sem_out[p])

    def drain_out(p):
        pltpu.make_async_copy(rows_v.at[p],
                              out_ref.at[pl.ds(0, T_PER_CHUNK)],
                              sem_out[p]).wait()

    @pl.when(lax.axis_index("s") == 0)
    def _():
        pltpu.async_copy(cent_ref, cent_sh, sem_cent)
        pltpu.async_copy(codes0_ref, codes_sh.at[0], sem_cent)
        pltpu.async_copy(codes1_ref, codes_sh.at[1], sem_cent)
        pltpu.make_async_copy(cent_ref, cent_sh, sem_cent).wait()
        pltpu.make_async_copy(codes0_ref, codes_sh.at[0], sem_cent).wait()
        pltpu.make_async_copy(codes1_ref, codes_sh.at[1], sem_cent).wait()
    plsc.subcore_barrier()
    fire_ids(0, 0)
    fire_ids(1, 1)
    wait_ids(0)
    fire_codes(0)

    def pair_body(kk, carry):
        for p in (0, 1):
            k = 2 * kk + p

            wait_codes(p)

            @pl.when(k < NCHUNK - 2)
            def _():
                fire_ids(k + 2, p)

            flat_compute(p)

            @pl.when(k >= 2)
            def _():
                drain_out(p)

            @pl.when(k >= 1)
            def _():
                drain_gathers(1 - p)
                fire_out(k - 1, 1 - p)

            fire_gathers(p)

            @pl.when(k < NCHUNK - 1)
            def _():
                wait_ids(1 - p)
                fire_codes(1 - p)
        return carry

    lax.fori_loop(0, NCHUNK // 2, pair_body, 0)
    drain_gathers(1)
    fire_out(NCHUNK - 1, 1)
    drain_out(0)
    drain_out(1)


@jax.jit
def _sc_call(ids1d, codes0, codes1, cent2d):
    mesh = plsc.VectorSubcoreMesh(core_axis_name="c", subcore_axis_name="s")
    f = pl.kernel(
        _body,
        out_type=jax.ShapeDtypeStruct((TOTAL // SUB_EMB, 128, SUB_EMB),
                                      jnp.float32),
        mesh=mesh,
        scratch_types=[
            pltpu.VMEM((2, CHUNK), jnp.int32),
            pltpu.VMEM((2, CHUNK), jnp.int32),
            pltpu.VMEM((2, CHUNK), jnp.int32),
            pltpu.VMEM((2, GROUPS, 128), jnp.int32),
            pltpu.VMEM((2, GROUPS, 128, SUB_EMB), jnp.float32),
            pltpu.VMEM_SHARED((PQ_M * VALS_PER_DIM, SUB_EMB), jnp.float32),
            pltpu.VMEM_SHARED((2, 100001), jnp.int32),
            pltpu.SemaphoreType.DMA,
            pltpu.SemaphoreType.DMA,
            pltpu.SemaphoreType.DMA,
            pltpu.SemaphoreType.DMA,
            pltpu.SemaphoreType.DMA,
            pltpu.SemaphoreType.DMA,
            pltpu.SemaphoreType.DMA,
            pltpu.SemaphoreType.DMA,
            pltpu.SemaphoreType.DMA,
        ],
        compiler_params=pltpu.CompilerParams(use_tc_tiling_on_sc=False,
                                             needs_layout_passes=False),
    )
    return f(ids1d, codes0, codes1, cent2d)


def kernel(input_ids, item_codes, centroids):
    ids1d = input_ids.reshape(TOTAL)
    cent2d = centroids.reshape(PQ_M * VALS_PER_DIM, SUB_EMB)
    c = item_codes
    w0 = c[:, 0] | (c[:, 1] << 8) | (c[:, 2] << 16) | (c[:, 3] << 24)
    w1 = c[:, 4] | (c[:, 5] << 8) | (c[:, 6] << 16) | (c[:, 7] << 24)
    out3d = _sc_call(ids1d, w0, w1, cent2d)
    return out3d.reshape(BATCH, SEQ_LEN, EMB)
